# baseline (speedup 1.0000x reference)
import functools

import jax
import jax.numpy as jnp
from jax import lax
from jax.experimental import pallas as pl
from jax.experimental.pallas import tpu as pltpu
from jax.experimental.pallas import tpu_sc as plsc

N_ = 10000
F_ = 320000
C_ = 128
CP_ = C_ // 2
NC_ = 2
NS_ = 16
NW_ = NC_ * NS_
CHUNK_ = 128
NFULL_ = 78
NEXTRA_ = (F_ - NW_ * NFULL_ * CHUNK_) // CHUNK_
ROWS_W_ = NFULL_ * CHUNK_

BT_ = 2000
NB_ = F_ // BT_


def _sc_body(inp_hbm, sw_hbm, i0_hbm, i1_hbm, i2_hbm, v2f_hbm,
             iv0, iv1, iv2, swv,
             ra0, ra1, ra2, rb0, rb1, rb2, fa, fb, sa, sb):
  wid = lax.axis_index("s") * NC_ + lax.axis_index("c")
  wbase = wid * ROWS_W_

  pltpu.sync_copy(i0_hbm.at[pl.ds(wbase, ROWS_W_)], iv0)
  pltpu.sync_copy(i1_hbm.at[pl.ds(wbase, ROWS_W_)], iv1)
  pltpu.sync_copy(i2_hbm.at[pl.ds(wbase, ROWS_W_)], iv2)
  pltpu.sync_copy(sw_hbm, swv)

  ngr = CP_ // 16
  w = [[(swv[k, pl.ds(g * 16, 16)], swv[k, pl.ds(CP_ + g * 16, 16)])
        for g in range(ngr)] for k in range(3)]

  def issue(bufs, sem, j):
    off = j * CHUNK_
    pltpu.async_copy(inp_hbm.at[iv0.at[pl.ds(off, CHUNK_)]], bufs[0], sem)
    pltpu.async_copy(inp_hbm.at[iv1.at[pl.ds(off, CHUNK_)]], bufs[1], sem)
    pltpu.async_copy(inp_hbm.at[iv2.at[pl.ds(off, CHUNK_)]], bufs[2], sem)

  def drain(bufs, sem, j):
    off = j * CHUNK_
    pltpu.make_async_copy(inp_hbm.at[iv0.at[pl.ds(off, CHUNK_)]], bufs[0],
                          sem).wait()
    pltpu.make_async_copy(inp_hbm.at[iv1.at[pl.ds(off, CHUNK_)]], bufs[1],
                          sem).wait()
    pltpu.make_async_copy(inp_hbm.at[iv2.at[pl.ds(off, CHUNK_)]], bufs[2],
                          sem).wait()

  mhi = jnp.int32(-65536)

  def combine(bufs, fout):
    def row(r, carry):
      for g in range(ngr):
        sl = pl.ds(g * 16, 16)
        xi = [bufs[k][r, sl] for k in range(3)]
        ev = [lax.bitcast_convert_type(lax.shift_left(x, 16), jnp.float32)
              for x in xi]
        od = [lax.bitcast_convert_type(jnp.bitwise_and(x, mhi), jnp.float32)
              for x in xi]
        oa = (ev[0] * w[0][g][0] + ev[1] * w[1][g][0] + ev[2] * w[2][g][0])
        ob = (od[0] * w[0][g][1] + od[1] * w[1][g][1] + od[2] * w[2][g][1])
        fout[r, pl.ds(g * 16, 16)] = oa
        fout[r, pl.ds(CP_ + g * 16, 16)] = ob
      return carry

    lax.fori_loop(0, CHUNK_, row, 0)

  def store(fout, base):
    pltpu.sync_copy(fout, v2f_hbm.at[pl.ds(base, CHUNK_)])

  bufs_a = (ra0, ra1, ra2)
  bufs_b = (rb0, rb1, rb2)

  issue(bufs_a, sa, 0)

  def body(i, carry):
    j0 = 2 * i
    issue(bufs_b, sb, j0 + 1)
    drain(bufs_a, sa, j0)
    combine(bufs_a, fa)
    store(fa, wbase + j0 * CHUNK_)

    @pl.when(j0 + 2 < NFULL_)
    def _():
      issue(bufs_a, sa, j0 + 2)

    drain(bufs_b, sb, j0 + 1)
    combine(bufs_b, fb)
    store(fb, wbase + (j0 + 1) * CHUNK_)
    return carry

  lax.fori_loop(0, NFULL_ // 2, body, 0)

  @pl.when(wid < NEXTRA_)
  def _():
    base = (NW_ * NFULL_ + wid) * CHUNK_
    pltpu.sync_copy(i0_hbm.at[pl.ds(base, CHUNK_)], iv0.at[pl.ds(0, CHUNK_)])
    pltpu.sync_copy(i1_hbm.at[pl.ds(base, CHUNK_)], iv1.at[pl.ds(0, CHUNK_)])
    pltpu.sync_copy(i2_hbm.at[pl.ds(base, CHUNK_)], iv2.at[pl.ds(0, CHUNK_)])
    issue(bufs_a, sa, 0)
    drain(bufs_a, sa, 0)
    combine(bufs_a, fa)
    store(fa, base)


@functools.lru_cache(maxsize=None)
def _get_sc_combine():
  return pl.kernel(
    out_type=jax.ShapeDtypeStruct((F_, C_), jnp.float32),
    mesh=plsc.VectorSubcoreMesh(core_axis_name="c", subcore_axis_name="s"),
    compiler_params=pltpu.CompilerParams(use_tc_tiling_on_sc=False),
    scratch_types=[
        pltpu.VMEM((ROWS_W_,), jnp.int32),
        pltpu.VMEM((ROWS_W_,), jnp.int32),
        pltpu.VMEM((ROWS_W_,), jnp.int32),
        pltpu.VMEM((8, C_), jnp.float32),
        pltpu.VMEM((CHUNK_, CP_), jnp.int32),
        pltpu.VMEM((CHUNK_, CP_), jnp.int32),
        pltpu.VMEM((CHUNK_, CP_), jnp.int32),
        pltpu.VMEM((CHUNK_, CP_), jnp.int32),
        pltpu.VMEM((CHUNK_, CP_), jnp.int32),
        pltpu.VMEM((CHUNK_, CP_), jnp.int32),
        pltpu.VMEM((CHUNK_, C_), jnp.float32),
        pltpu.VMEM((CHUNK_, C_), jnp.float32),
        pltpu.SemaphoreType.DMA,
        pltpu.SemaphoreType.DMA,
    ],
  )(_sc_body)


def _ab_body(v2f, dwb, bb, gb, out, acc):
  p = pl.program_id(0)
  r = jnp.dot(v2f[...], dwb[...], preferred_element_type=jnp.float32)
  r = jnp.maximum(r + bb[0, :][None, :], 0.0)

  @pl.when(p == 0)
  def _():
    s = jnp.sum(r, axis=0)
    s2 = jnp.sum(r * r, axis=0)
    upd = jnp.concatenate(
        [s[None, :], s2[None, :], jnp.zeros((6, C_), jnp.float32)], axis=0)

    @pl.when(pl.program_id(1) == 0)
    def _():
      acc[...] = upd

    @pl.when(pl.program_id(1) != 0)
    def _():
      acc[...] = acc[...] + upd

  @pl.when(p == 1)
  def _():
    mean = acc[0, :] / F_
    var = acc[1, :] / F_ - mean * mean
    inv = gb[0, :] / jnp.sqrt(var + 1e-5)
    out[...] = (r - mean[None, :]) * inv[None, :] + gb[1, :][None, :]


def kernel(inputs, face, spatial_weights, depth_weights, biases,
           bn_gamma, bn_beta):
  face32 = face.astype(jnp.int32)
  ft = face32.T
  i0 = ft[0]
  i1 = ft[1]
  i2 = ft[2]

  inb = inputs.astype(jnp.bfloat16)
  in_i32 = lax.bitcast_convert_type(inb.reshape(N_, CP_, 2), jnp.int32)

  sw2 = spatial_weights[:, :, 0]
  sw_perm = jnp.concatenate([sw2[:, 0::2], sw2[:, 1::2]], axis=1)
  sw8 = jnp.pad(sw_perm, ((0, 5), (0, 0)))

  dwp = jnp.concatenate([depth_weights[0::2, :], depth_weights[1::2, :]],
                        axis=0)

  bb8 = jnp.pad(biases, ((0, 7), (0, 0)))
  gb8 = jnp.pad(jnp.stack([bn_gamma, bn_beta]), ((0, 6), (0, 0)))

  v2f = _get_sc_combine()(in_i32, sw8, i0, i1, i2)

  out = pl.pallas_call(
      _ab_body,
      grid=(2, NB_),
      in_specs=[
          pl.BlockSpec((BT_, C_), lambda p, i: (i, 0)),
          pl.BlockSpec((C_, C_), lambda p, i: (0, 0)),
          pl.BlockSpec((8, C_), lambda p, i: (0, 0)),
          pl.BlockSpec((8, C_), lambda p, i: (0, 0)),
      ],
      out_specs=pl.BlockSpec((BT_, C_),
                             lambda p, i: (jnp.where(p == 1, i, 0), 0)),
      out_shape=jax.ShapeDtypeStruct((F_, C_), jnp.float32),
      scratch_shapes=[pltpu.VMEM((8, C_), jnp.float32)],
  )(v2f, dwp, bb8, gb8)

  return out


# speedup vs baseline: 1.3603x; 1.3603x over previous
import functools

import jax
import jax.numpy as jnp
from jax import lax
from jax.experimental import pallas as pl
from jax.experimental.pallas import tpu as pltpu
from jax.experimental.pallas import tpu_sc as plsc

N_ = 10000
F_ = 320000
C_ = 128
NC_ = 2
NS_ = 16
NW_ = NC_ * NS_
CHUNK_ = 128
NFULL_ = 78
NEXTRA_ = (F_ - NW_ * NFULL_ * CHUNK_) // CHUNK_
ROWS_W_ = NFULL_ * CHUNK_

BT_ = 2000
NB_ = F_ // BT_


def _c_body(inp, sw, t):
  x = inp[...]
  t[pl.ds(0, N_), :] = x * sw[0, :][None, :]
  t[pl.ds(N_, N_), :] = x * sw[1, :][None, :]
  t[pl.ds(2 * N_, N_), :] = x * sw[2, :][None, :]


def _sc_body(t_hbm, i0_hbm, i1_hbm, i2_hbm, v2f_hbm,
             iv0, iv1, iv2,
             ra0, ra1, ra2, rb0, rb1, rb2, sa, sb):
  wid = lax.axis_index("s") * NC_ + lax.axis_index("c")
  wbase = wid * ROWS_W_

  pltpu.sync_copy(i0_hbm.at[pl.ds(wbase, ROWS_W_)], iv0)
  pltpu.sync_copy(i1_hbm.at[pl.ds(wbase, ROWS_W_)], iv1)
  pltpu.sync_copy(i2_hbm.at[pl.ds(wbase, ROWS_W_)], iv2)

  def issue(bufs, sem, j):
    off = j * CHUNK_
    pltpu.async_copy(t_hbm.at[iv0.at[pl.ds(off, CHUNK_)]], bufs[0], sem)
    pltpu.async_copy(t_hbm.at[iv1.at[pl.ds(off, CHUNK_)]], bufs[1], sem)
    pltpu.async_copy(t_hbm.at[iv2.at[pl.ds(off, CHUNK_)]], bufs[2], sem)

  def drain(bufs, sem, j):
    off = j * CHUNK_
    pltpu.make_async_copy(t_hbm.at[iv0.at[pl.ds(off, CHUNK_)]], bufs[0],
                          sem).wait()
    pltpu.make_async_copy(t_hbm.at[iv1.at[pl.ds(off, CHUNK_)]], bufs[1],
                          sem).wait()
    pltpu.make_async_copy(t_hbm.at[iv2.at[pl.ds(off, CHUNK_)]], bufs[2],
                          sem).wait()

  def combine(bufs):
    def row(r, carry):
      for s in range(C_ // 16):
        sl = pl.ds(s * 16, 16)
        bufs[0][r, sl] = bufs[0][r, sl] + bufs[1][r, sl] + bufs[2][r, sl]
      return carry

    lax.fori_loop(0, CHUNK_, row, 0)

  def store(bufs, base):
    pltpu.sync_copy(bufs[0], v2f_hbm.at[pl.ds(base, CHUNK_)])

  bufs_a = (ra0, ra1, ra2)
  bufs_b = (rb0, rb1, rb2)

  issue(bufs_a, sa, 0)

  def body(i, carry):
    j0 = 2 * i
    issue(bufs_b, sb, j0 + 1)
    drain(bufs_a, sa, j0)
    combine(bufs_a)
    store(bufs_a, wbase + j0 * CHUNK_)

    @pl.when(j0 + 2 < NFULL_)
    def _():
      issue(bufs_a, sa, j0 + 2)

    drain(bufs_b, sb, j0 + 1)
    combine(bufs_b)
    store(bufs_b, wbase + (j0 + 1) * CHUNK_)
    return carry

  lax.fori_loop(0, NFULL_ // 2, body, 0)

  @pl.when(wid < NEXTRA_)
  def _():
    base = (NW_ * NFULL_ + wid) * CHUNK_
    pltpu.sync_copy(i0_hbm.at[pl.ds(base, CHUNK_)], iv0.at[pl.ds(0, CHUNK_)])
    pltpu.sync_copy(i1_hbm.at[pl.ds(base, CHUNK_)], iv1.at[pl.ds(0, CHUNK_)])
    pltpu.sync_copy(i2_hbm.at[pl.ds(base, CHUNK_)], iv2.at[pl.ds(0, CHUNK_)])
    issue(bufs_a, sa, 0)
    drain(bufs_a, sa, 0)
    combine(bufs_a)
    store(bufs_a, base)


@functools.lru_cache(maxsize=None)
def _get_sc_combine():
  return pl.kernel(
    out_type=jax.ShapeDtypeStruct((F_, C_), jnp.float32),
    mesh=plsc.VectorSubcoreMesh(core_axis_name="c", subcore_axis_name="s"),
    scratch_types=[
        pltpu.VMEM((ROWS_W_,), jnp.int32),
        pltpu.VMEM((ROWS_W_,), jnp.int32),
        pltpu.VMEM((ROWS_W_,), jnp.int32),
        pltpu.VMEM((CHUNK_, C_), jnp.float32),
        pltpu.VMEM((CHUNK_, C_), jnp.float32),
        pltpu.VMEM((CHUNK_, C_), jnp.float32),
        pltpu.VMEM((CHUNK_, C_), jnp.float32),
        pltpu.VMEM((CHUNK_, C_), jnp.float32),
        pltpu.VMEM((CHUNK_, C_), jnp.float32),
        pltpu.SemaphoreType.DMA,
        pltpu.SemaphoreType.DMA,
    ],
  )(_sc_body)


def _a_body(v2f, dw, bb, stats):
  acc = jnp.dot(v2f[...], dw[...], preferred_element_type=jnp.float32)
  acc = acc + bb[0, :][None, :]
  r = jnp.maximum(acc, 0.0)
  s = jnp.sum(r, axis=0)
  s2 = jnp.sum(r * r, axis=0)
  upd = jnp.concatenate(
      [s[None, :], s2[None, :], jnp.zeros((6, C_), jnp.float32)], axis=0)

  @pl.when(pl.program_id(0) == 0)
  def _():
    stats[...] = upd

  @pl.when(pl.program_id(0) != 0)
  def _():
    stats[...] = stats[...] + upd


def _b_body(v2f, dw, bb, stats, gb, out):
  acc = jnp.dot(v2f[...], dw[...], preferred_element_type=jnp.float32)
  acc = acc + bb[0, :][None, :]
  r = jnp.maximum(acc, 0.0)
  mean = stats[0, :] / F_
  var = stats[1, :] / F_ - mean * mean
  inv = gb[0, :] / jnp.sqrt(var + 1e-5)
  out[...] = (r - mean[None, :]) * inv[None, :] + gb[1, :][None, :]


def kernel(inputs, face, spatial_weights, depth_weights, biases,
           bn_gamma, bn_beta):
  face32 = face.astype(jnp.int32)
  ft = face32.T
  i0 = ft[0]
  i1 = ft[1] + N_
  i2 = ft[2] + 2 * N_

  sw8 = jnp.pad(spatial_weights[:, :, 0], ((0, 5), (0, 0)))
  bb8 = jnp.pad(biases, ((0, 7), (0, 0)))
  gb8 = jnp.pad(jnp.stack([bn_gamma, bn_beta]), ((0, 6), (0, 0)))

  t = pl.pallas_call(
      _c_body,
      in_specs=[
          pl.BlockSpec((N_, C_), lambda: (0, 0)),
          pl.BlockSpec((8, C_), lambda: (0, 0)),
      ],
      out_specs=pl.BlockSpec((3 * N_, C_), lambda: (0, 0)),
      out_shape=jax.ShapeDtypeStruct((3 * N_, C_), jnp.float32),
  )(inputs, sw8)

  v2f = _get_sc_combine()(t, i0, i1, i2)

  stats = pl.pallas_call(
      _a_body,
      grid=(NB_,),
      in_specs=[
          pl.BlockSpec((BT_, C_), lambda i: (i, 0)),
          pl.BlockSpec((C_, C_), lambda i: (0, 0)),
          pl.BlockSpec((8, C_), lambda i: (0, 0)),
      ],
      out_specs=pl.BlockSpec((8, C_), lambda i: (0, 0)),
      out_shape=jax.ShapeDtypeStruct((8, C_), jnp.float32),
  )(v2f, depth_weights, bb8)

  out = pl.pallas_call(
      _b_body,
      grid=(NB_,),
      in_specs=[
          pl.BlockSpec((BT_, C_), lambda i: (i, 0)),
          pl.BlockSpec((C_, C_), lambda i: (0, 0)),
          pl.BlockSpec((8, C_), lambda i: (0, 0)),
          pl.BlockSpec((8, C_), lambda i: (0, 0)),
          pl.BlockSpec((8, C_), lambda i: (0, 0)),
      ],
      out_specs=pl.BlockSpec((BT_, C_), lambda i: (i, 0)),
      out_shape=jax.ShapeDtypeStruct((F_, C_), jnp.float32),
  )(v2f, depth_weights, bb8, stats, gb8)

  return out


# speedup vs baseline: 1.4247x; 1.0474x over previous
import functools

import jax
import jax.numpy as jnp
from jax import lax
from jax.experimental import pallas as pl
from jax.experimental.pallas import tpu as pltpu
from jax.experimental.pallas import tpu_sc as plsc

N_ = 10000
F_ = 320000
C_ = 128
NC_ = 2
NS_ = 16
NW_ = NC_ * NS_
CHUNK_ = 128
FH_ = F_ // 2
NFULL_ = 39
NEXTRA_ = (FH_ - NW_ * NFULL_ * CHUNK_) // CHUNK_
ROWS_W_ = NFULL_ * CHUNK_

BT_ = 2000
NB_ = F_ // BT_
NBH_ = FH_ // BT_


def _c_body(inp, sw, t):
  x = inp[...]
  t[pl.ds(0, N_), :] = x * sw[0, :][None, :]
  t[pl.ds(N_, N_), :] = x * sw[1, :][None, :]
  t[pl.ds(2 * N_, N_), :] = x * sw[2, :][None, :]


def _sc_body(base0, t_hbm, i0_hbm, i1_hbm, i2_hbm, v2f_hbm,
             iv0, iv1, iv2,
             ra0, ra1, ra2, rb0, rb1, rb2, sa, sb):
  wid = lax.axis_index("s") * NC_ + lax.axis_index("c")
  wbase = wid * ROWS_W_
  gbase = base0 + wbase

  pltpu.sync_copy(i0_hbm.at[pl.ds(gbase, ROWS_W_)], iv0)
  pltpu.sync_copy(i1_hbm.at[pl.ds(gbase, ROWS_W_)], iv1)
  pltpu.sync_copy(i2_hbm.at[pl.ds(gbase, ROWS_W_)], iv2)

  def issue(bufs, sem, j):
    off = j * CHUNK_
    pltpu.async_copy(t_hbm.at[iv0.at[pl.ds(off, CHUNK_)]], bufs[0], sem)
    pltpu.async_copy(t_hbm.at[iv1.at[pl.ds(off, CHUNK_)]], bufs[1], sem)
    pltpu.async_copy(t_hbm.at[iv2.at[pl.ds(off, CHUNK_)]], bufs[2], sem)

  def drain(bufs, sem, j):
    off = j * CHUNK_
    pltpu.make_async_copy(t_hbm.at[iv0.at[pl.ds(off, CHUNK_)]], bufs[0],
                          sem).wait()
    pltpu.make_async_copy(t_hbm.at[iv1.at[pl.ds(off, CHUNK_)]], bufs[1],
                          sem).wait()
    pltpu.make_async_copy(t_hbm.at[iv2.at[pl.ds(off, CHUNK_)]], bufs[2],
                          sem).wait()

  def combine(bufs):
    def row(r, carry):
      for s in range(C_ // 16):
        sl = pl.ds(s * 16, 16)
        bufs[0][r, sl] = bufs[0][r, sl] + bufs[1][r, sl] + bufs[2][r, sl]
      return carry

    lax.fori_loop(0, CHUNK_, row, 0)

  def store(bufs, base):
    pltpu.sync_copy(bufs[0], v2f_hbm.at[pl.ds(base, CHUNK_)])

  bufs_a = (ra0, ra1, ra2)
  bufs_b = (rb0, rb1, rb2)

  issue(bufs_a, sa, 0)

  def body(i, carry):
    j0 = 2 * i

    @pl.when(j0 + 1 < NFULL_)
    def _():
      issue(bufs_b, sb, j0 + 1)

    drain(bufs_a, sa, j0)
    combine(bufs_a)
    store(bufs_a, wbase + j0 * CHUNK_)

    @pl.when(j0 + 2 < NFULL_)
    def _():
      issue(bufs_a, sa, j0 + 2)

    @pl.when(j0 + 1 < NFULL_)
    def _():
      drain(bufs_b, sb, j0 + 1)
      combine(bufs_b)
      store(bufs_b, wbase + (j0 + 1) * CHUNK_)
    return carry

  lax.fori_loop(0, (NFULL_ + 1) // 2, body, 0)

  @pl.when(wid < NEXTRA_)
  def _():
    base = (NW_ * NFULL_ + wid) * CHUNK_
    pltpu.sync_copy(i0_hbm.at[pl.ds(base0 + base, CHUNK_)],
                    iv0.at[pl.ds(0, CHUNK_)])
    pltpu.sync_copy(i1_hbm.at[pl.ds(base0 + base, CHUNK_)],
                    iv1.at[pl.ds(0, CHUNK_)])
    pltpu.sync_copy(i2_hbm.at[pl.ds(base0 + base, CHUNK_)],
                    iv2.at[pl.ds(0, CHUNK_)])
    issue(bufs_a, sa, 0)
    drain(bufs_a, sa, 0)
    combine(bufs_a)
    store(bufs_a, base)


@functools.lru_cache(maxsize=None)
def _get_sc_combine(base0):
  return pl.kernel(
    out_type=jax.ShapeDtypeStruct((FH_, C_), jnp.float32),
    mesh=plsc.VectorSubcoreMesh(core_axis_name="c", subcore_axis_name="s"),
    scratch_types=[
        pltpu.VMEM((ROWS_W_,), jnp.int32),
        pltpu.VMEM((ROWS_W_,), jnp.int32),
        pltpu.VMEM((ROWS_W_,), jnp.int32),
        pltpu.VMEM((CHUNK_, C_), jnp.float32),
        pltpu.VMEM((CHUNK_, C_), jnp.float32),
        pltpu.VMEM((CHUNK_, C_), jnp.float32),
        pltpu.VMEM((CHUNK_, C_), jnp.float32),
        pltpu.VMEM((CHUNK_, C_), jnp.float32),
        pltpu.VMEM((CHUNK_, C_), jnp.float32),
        pltpu.SemaphoreType.DMA,
        pltpu.SemaphoreType.DMA,
    ],
  )(functools.partial(_sc_body, base0))


def _a_body(v2f, dw, bb, stats):
  acc = jnp.dot(v2f[...], dw[...], preferred_element_type=jnp.float32)
  acc = acc + bb[0, :][None, :]
  r = jnp.maximum(acc, 0.0)
  s = jnp.sum(r, axis=0)
  s2 = jnp.sum(r * r, axis=0)
  upd = jnp.concatenate(
      [s[None, :], s2[None, :], jnp.zeros((6, C_), jnp.float32)], axis=0)

  @pl.when(pl.program_id(0) == 0)
  def _():
    stats[...] = upd

  @pl.when(pl.program_id(0) != 0)
  def _():
    stats[...] = stats[...] + upd


def _b_body(vh1, vh2, dw, bb, st1, st2, gb, out):
  i = pl.program_id(0)
  x = jnp.where(i < NBH_, vh1[...], vh2[...])
  acc = jnp.dot(x, dw[...], preferred_element_type=jnp.float32)
  acc = acc + bb[0, :][None, :]
  r = jnp.maximum(acc, 0.0)
  s = st1[0, :] + st2[0, :]
  s2 = st1[1, :] + st2[1, :]
  mean = s / F_
  var = s2 / F_ - mean * mean
  inv = gb[0, :] / jnp.sqrt(var + 1e-5)
  out[...] = (r - mean[None, :]) * inv[None, :] + gb[1, :][None, :]


def kernel(inputs, face, spatial_weights, depth_weights, biases,
           bn_gamma, bn_beta):
  face32 = face.astype(jnp.int32)
  ft = face32.T
  i0 = ft[0]
  i1 = ft[1] + N_
  i2 = ft[2] + 2 * N_

  sw8 = jnp.pad(spatial_weights[:, :, 0], ((0, 5), (0, 0)))
  bb8 = jnp.pad(biases, ((0, 7), (0, 0)))
  gb8 = jnp.pad(jnp.stack([bn_gamma, bn_beta]), ((0, 6), (0, 0)))

  t = pl.pallas_call(
      _c_body,
      in_specs=[
          pl.BlockSpec((N_, C_), lambda: (0, 0)),
          pl.BlockSpec((8, C_), lambda: (0, 0)),
      ],
      out_specs=pl.BlockSpec((3 * N_, C_), lambda: (0, 0)),
      out_shape=jax.ShapeDtypeStruct((3 * N_, C_), jnp.float32),
  )(inputs, sw8)

  vh1 = _get_sc_combine(0)(t, i0, i1, i2)
  vh2 = _get_sc_combine(FH_)(t, i0, i1, i2)

  def run_a(vh):
    return pl.pallas_call(
        _a_body,
        grid=(NBH_,),
        in_specs=[
            pl.BlockSpec((BT_, C_), lambda i: (i, 0)),
            pl.BlockSpec((C_, C_), lambda i: (0, 0)),
            pl.BlockSpec((8, C_), lambda i: (0, 0)),
        ],
        out_specs=pl.BlockSpec((8, C_), lambda i: (0, 0)),
        out_shape=jax.ShapeDtypeStruct((8, C_), jnp.float32),
    )(vh, depth_weights, bb8)

  st1 = run_a(vh1)
  st2 = run_a(vh2)

  out = pl.pallas_call(
      _b_body,
      grid=(NB_,),
      in_specs=[
          pl.BlockSpec((BT_, C_),
                       lambda i: (jnp.minimum(i, NBH_ - 1), 0)),
          pl.BlockSpec((BT_, C_),
                       lambda i: (jnp.maximum(i - NBH_, 0), 0)),
          pl.BlockSpec((C_, C_), lambda i: (0, 0)),
          pl.BlockSpec((8, C_), lambda i: (0, 0)),
          pl.BlockSpec((8, C_), lambda i: (0, 0)),
          pl.BlockSpec((8, C_), lambda i: (0, 0)),
          pl.BlockSpec((8, C_), lambda i: (0, 0)),
      ],
      out_specs=pl.BlockSpec((BT_, C_), lambda i: (i, 0)),
      out_shape=jax.ShapeDtypeStruct((F_, C_), jnp.float32),
  )(vh1, vh2, depth_weights, bb8, st1, st2, gb8)

  return out
